# baseline (speedup 1.0000x reference)
import functools

import jax
import jax.numpy as jnp
from jax import lax
from jax.experimental import pallas as pl
from jax.experimental.pallas import tpu as pltpu
from jax.experimental.pallas import tpu_sc as plsc

_INFO = plsc.get_sparse_core_info()
_NC = _INFO.num_cores

_CHUNK_ROWS = 512
_NBUF = 4


def _make_scs_copy(rows, dim, dtype):
    rows_per_c = rows // _NC
    nchunk = rows_per_c // _CHUNK_ROWS
    mesh = plsc.ScalarSubcoreMesh(axis_name="c", num_cores=_NC)

    @functools.partial(
        pl.kernel,
        mesh=mesh,
        out_type=jax.ShapeDtypeStruct((rows, dim), dtype),
        scratch_types=(
            [
                pltpu.MemorySpace.VMEM_SHARED((_CHUNK_ROWS, dim), dtype)
                for _ in range(_NBUF)
            ]
            + [
                pltpu.SemaphoreType.DMA((nchunk,)),
                pltpu.SemaphoreType.DMA((nchunk,)),
            ]
        ),
    )
    def scs_copy(table_hbm, out_hbm, *rest):
        bufs, (in_sems, out_sems) = rest[:_NBUF], rest[_NBUF:]
        cid = lax.axis_index("c")
        base = cid * rows_per_c
        loads = [
            pltpu.make_async_copy(
                table_hbm.at[pl.ds(base + j * _CHUNK_ROWS, _CHUNK_ROWS)],
                bufs[j % _NBUF],
                in_sems.at[j],
            )
            for j in range(nchunk)
        ]
        stores = [
            pltpu.make_async_copy(
                bufs[j % _NBUF],
                out_hbm.at[pl.ds(base + j * _CHUNK_ROWS, _CHUNK_ROWS)],
                out_sems.at[j],
            )
            for j in range(nchunk)
        ]
        for j in range(min(_NBUF, nchunk)):
            loads[j].start()
        for j in range(nchunk):
            loads[j].wait()
            stores[j].start()
            if j + _NBUF < nchunk:
                stores[j].wait()
                loads[j + _NBUF].start()
        for j in range(max(0, nchunk - _NBUF), nchunk):
            stores[j].wait()

    return scs_copy


def kernel(x, table):
    del x
    rows, dim = table.shape
    out = _make_scs_copy(rows, dim, table.dtype)(table)
    return out[None]


# speedup vs baseline: 1.0437x; 1.0437x over previous
import functools

import jax
import jax.numpy as jnp
from jax import lax
from jax.experimental import pallas as pl
from jax.experimental.pallas import tpu as pltpu
from jax.experimental.pallas import tpu_sc as plsc

_INFO = plsc.get_sparse_core_info()
_NC, _NS = _INFO.num_cores, _INFO.num_subcores
_NW = _NC * _NS

_CHUNK_ROWS = 32
_NBUF = 4


def _make_sc_copy(rows, dim, dtype):
    rows_per_w = rows // _NW
    nchunk = rows_per_w // _CHUNK_ROWS
    mesh = plsc.VectorSubcoreMesh(core_axis_name="c", subcore_axis_name="s")

    @functools.partial(
        pl.kernel,
        mesh=mesh,
        out_type=jax.ShapeDtypeStruct((rows, dim), dtype),
        scratch_types=(
            [pltpu.VMEM((_CHUNK_ROWS, dim), dtype) for _ in range(_NBUF)]
            + [
                pltpu.SemaphoreType.DMA((nchunk,)),
                pltpu.SemaphoreType.DMA((nchunk,)),
            ]
        ),
    )
    def sc_copy(table_hbm, out_hbm, *rest):
        bufs, (in_sems, out_sems) = rest[:_NBUF], rest[_NBUF:]
        wid = lax.axis_index("s") * _NC + lax.axis_index("c")
        base = wid * rows_per_w
        loads = [
            pltpu.make_async_copy(
                table_hbm.at[pl.ds(base + j * _CHUNK_ROWS, _CHUNK_ROWS)],
                bufs[j % _NBUF],
                in_sems.at[j],
            )
            for j in range(nchunk)
        ]
        stores = [
            pltpu.make_async_copy(
                bufs[j % _NBUF],
                out_hbm.at[pl.ds(base + j * _CHUNK_ROWS, _CHUNK_ROWS)],
                out_sems.at[j],
            )
            for j in range(nchunk)
        ]
        for j in range(min(_NBUF, nchunk)):
            loads[j].start()
        for j in range(nchunk):
            loads[j].wait()
            stores[j].start()
            if j + _NBUF < nchunk:
                stores[j].wait()
                loads[j + _NBUF].start()
        for j in range(max(0, nchunk - _NBUF), nchunk):
            stores[j].wait()

    return sc_copy


def kernel(x, table):
    del x
    rows, dim = table.shape
    out = _make_sc_copy(rows, dim, table.dtype)(table)
    return out[None]
